# Initial kernel scaffold; baseline (speedup 1.0000x reference)
import jax
import jax.numpy as jnp
from jax.experimental import pallas as pl


def kernel(x, kernel_rff, W0, b0, W1, b1, W2, b2, W3, b3, W4, b4):
    raise NotImplementedError("write your pallas kernel here")


# speedup vs baseline: 1.0752x; 1.0752x over previous
import jax
import jax.numpy as jnp
from jax.experimental import pallas as pl
from jax.experimental.pallas import tpu as pltpu

_BLK = 2048


def _fused_mlp_kernel(x_ref, kt_ref, w0_ref, b0_ref, w1_ref, b1_ref,
                      w2_ref, b2_ref, w3_ref, b3_ref, w4_ref, b4_ref, y_ref):
    x = x_ref[...]
    xr = jnp.dot(x, kt_ref[...], preferred_element_type=jnp.float32)
    feats = jnp.concatenate((jnp.cos(xr), jnp.sin(xr)), axis=1)
    h = feats.astype(jnp.bfloat16)
    h = jnp.tanh(jnp.dot(h, w0_ref[...], preferred_element_type=jnp.float32)
                 + b0_ref[...]).astype(jnp.bfloat16)
    h = jnp.tanh(jnp.dot(h, w1_ref[...], preferred_element_type=jnp.float32)
                 + b1_ref[...]).astype(jnp.bfloat16)
    h = jnp.tanh(jnp.dot(h, w2_ref[...], preferred_element_type=jnp.float32)
                 + b2_ref[...]).astype(jnp.bfloat16)
    h = jnp.tanh(jnp.dot(h, w3_ref[...], preferred_element_type=jnp.float32)
                 + b3_ref[...]).astype(jnp.bfloat16)
    y_ref[...] = (jnp.dot(h, w4_ref[...], preferred_element_type=jnp.float32)
                  + b4_ref[...])


def kernel(x, kernel_rff, W0, b0, W1, b1, W2, b2, W3, b3, W4, b4):
    n = x.shape[0]
    kt = kernel_rff.T
    w0 = W0.T.astype(jnp.bfloat16)
    w1 = W1.T.astype(jnp.bfloat16)
    w2 = W2.T.astype(jnp.bfloat16)
    w3 = W3.T.astype(jnp.bfloat16)
    w4 = W4.T.astype(jnp.bfloat16)
    b0r, b1r, b2r, b3r = (b.reshape(1, -1) for b in (b0, b1, b2, b3))
    b4r = b4.reshape(1, -1)

    grid = (n // _BLK,)
    row = lambda i: (i, 0)
    rep = lambda i: (0, 0)

    y = pl.pallas_call(
        _fused_mlp_kernel,
        grid=grid,
        in_specs=[
            pl.BlockSpec((_BLK, 3), row),
            pl.BlockSpec((3, 128), rep),
            pl.BlockSpec((256, 1024), rep),
            pl.BlockSpec((1, 1024), rep),
            pl.BlockSpec((1024, 1024), rep),
            pl.BlockSpec((1, 1024), rep),
            pl.BlockSpec((1024, 1024), rep),
            pl.BlockSpec((1, 1024), rep),
            pl.BlockSpec((1024, 1024), rep),
            pl.BlockSpec((1, 1024), rep),
            pl.BlockSpec((1024, 3), rep),
            pl.BlockSpec((1, 3), rep),
        ],
        out_specs=pl.BlockSpec((_BLK, 3), row),
        out_shape=jax.ShapeDtypeStruct((n, 3), jnp.float32),
        compiler_params=pltpu.CompilerParams(
            dimension_semantics=("arbitrary",),
        ),
    )(x, kt, w0, b0r, w1, b1r, w2, b2r, w3, b3r, w4, b4r)
    return y
